# baseline (speedup 1.0000x reference)
import functools

import jax
import jax.numpy as jnp
from jax import lax
from jax.experimental import pallas as pl
from jax.experimental.pallas import tpu as pltpu
from jax.experimental.pallas import tpu_sc as plsc

N = 10000
D = 128
NC = 2
NS = 16
NW = NC * NS
SUB = 128
LANES = 16
N_PAD = 10240
ROWS_PER_TILE = N_PAD // NS
ZROWS = 128


def _sc_aggregate(x, src, dst, w, n_sub):
  mesh = plsc.VectorSubcoreMesh(
      core_axis_name="c", subcore_axis_name="s", num_cores=NC, num_subcores=NS
  )

  @functools.partial(
      pl.kernel,
      out_type=jax.ShapeDtypeStruct((NC, N_PAD, D), jnp.float32),
      mesh=mesh,
      scratch_types=[
          pltpu.VMEM((1, SUB), jnp.int32),
          pltpu.VMEM((1, SUB), jnp.int32),
          pltpu.VMEM((SUB,), jnp.float32),
          pltpu.VMEM((SUB, D), jnp.float32),
          pltpu.VMEM((ZROWS, D), jnp.float32),
          pltpu.VMEM_SHARED((N_PAD, D), jnp.float32),
          pltpu.SemaphoreType.DMA,
          pltpu.SemaphoreType.DMA,
      ],
  )
  def agg(x_hbm, src_hbm, dst_hbm, w_hbm, out_hbm,
          src_c, dst_c, w_c, rows, zbuf, acc, sem, semi):
    cid = lax.axis_index("c")
    sid = lax.axis_index("s")
    wid = cid * NS + sid

    zero16 = jnp.zeros((LANES,), jnp.float32)

    def zero_row(r, carry):
      for c in range(D // LANES):
        zbuf[r, pl.ds(c * LANES, LANES)] = zero16
      return carry

    lax.fori_loop(0, ZROWS, zero_row, 0)
    base = sid * ROWS_PER_TILE
    for k in range(ROWS_PER_TILE // ZROWS):
      pltpu.sync_copy(zbuf, acc.at[pl.ds(base + k * ZROWS, ZROWS)])
    plsc.subcore_barrier()

    def body(j, carry):
      cp1 = pltpu.async_copy(src_hbm.at[wid, j], src_c.at[0], semi)
      cp2 = pltpu.async_copy(dst_hbm.at[wid, j], dst_c.at[0], semi)
      cp3 = pltpu.async_copy(w_hbm.at[wid, pl.ds(j * SUB, SUB)], w_c, semi)
      cp1.wait()
      cp2.wait()
      cp3.wait()
      pltpu.async_copy(x_hbm.at[src_c.at[0]], rows, sem).wait()

      def scale16(i16, c2):
        w16 = w_c[pl.ds(i16 * LANES, LANES)]
        for bb in range(LANES):
          wspl = lax.gather(
              w16,
              jnp.full((LANES, 1), bb, jnp.int32),
              lax.GatherDimensionNumbers(
                  offset_dims=(), collapsed_slice_dims=(0,),
                  start_index_map=(0,)),
              slice_sizes=(1,),
              mode=lax.GatherScatterMode.PROMISE_IN_BOUNDS,
          )
          row = i16 * LANES + bb
          for c in range(D // LANES):
            rows[row, pl.ds(c * LANES, LANES)] = (
                rows[row, pl.ds(c * LANES, LANES)] * wspl
            )
        return c2

      lax.fori_loop(0, SUB // LANES, scale16, 0)
      pltpu.sync_copy(rows, acc.at[dst_c.at[0]], add=True)
      return carry

    lax.fori_loop(0, n_sub, body, 0)
    plsc.subcore_barrier()

    for k in range(ROWS_PER_TILE // ZROWS):
      pltpu.sync_copy(acc.at[pl.ds(base + k * ZROWS, ZROWS)], zbuf)
      pltpu.sync_copy(zbuf, out_hbm.at[cid, pl.ds(base + k * ZROWS, ZROWS)])

  return agg(x, src, dst, w)


def _tc_finish(p, W):
  blk = 1000
  grid = (N // blk,)

  def body(p_ref, w_ref, o_ref):
    a = p_ref[0] + p_ref[1]
    h = jnp.dot(a, w_ref[...], preferred_element_type=jnp.float32)
    o_ref[...] = jnp.maximum(h, 0.0)

  return pl.pallas_call(
      body,
      grid=grid,
      in_specs=[
          pl.BlockSpec((NC, blk, D), lambda i: (0, i, 0)),
          pl.BlockSpec((D, D), lambda i: (0, 0)),
      ],
      out_specs=pl.BlockSpec((blk, D), lambda i: (i, 0)),
      out_shape=jax.ShapeDtypeStruct((N, D), jnp.float32),
  )(p, W)


@jax.jit
def kernel(x, edge_index, edge_weight, W):
  src = edge_index[0]
  dst = edge_index[1]
  e = src.shape[0]
  n_sub = -(-e // (NW * SUB))
  e_pad = NW * SUB * n_sub
  pad = e_pad - e
  src = jnp.concatenate([src, jnp.zeros((pad,), jnp.int32)]).reshape(NW, n_sub, SUB)
  dst = jnp.concatenate([dst, jnp.zeros((pad,), jnp.int32)]).reshape(NW, n_sub, SUB)
  w = jnp.concatenate([edge_weight, jnp.zeros((pad,), jnp.float32)]).reshape(
      NW, n_sub * SUB
  )
  p = _sc_aggregate(x, src, dst, w, n_sub)
  return _tc_finish(p, W)


# speedup vs baseline: 1.0006x; 1.0006x over previous
import functools

import jax
import jax.numpy as jnp
from jax import lax
from jax.experimental import pallas as pl
from jax.experimental.pallas import tpu as pltpu
from jax.experimental.pallas import tpu_sc as plsc

N = 10000
D = 128
NC = 2
NS = 16
NW = NC * NS
SUB = 128
LANES = 16
N_PAD = 10240
ROWS_PER_TILE = N_PAD // NS
ZROWS = 128


def _sc_aggregate(x, src, dst, w, n_sub):
  mesh = plsc.VectorSubcoreMesh(
      core_axis_name="c", subcore_axis_name="s", num_cores=NC, num_subcores=NS
  )

  @functools.partial(
      pl.kernel,
      out_type=jax.ShapeDtypeStruct((NC, N_PAD, D), jnp.float32),
      mesh=mesh,
      scratch_types=[
          pltpu.VMEM((1, SUB), jnp.int32),
          pltpu.VMEM((1, SUB), jnp.int32),
          pltpu.VMEM((SUB,), jnp.float32),
          pltpu.VMEM((SUB, D), jnp.float32),
          pltpu.VMEM((ZROWS, D), jnp.float32),
          pltpu.VMEM_SHARED((N_PAD, D), jnp.float32),
          pltpu.SemaphoreType.DMA,
          pltpu.SemaphoreType.DMA,
      ],
  )
  def agg(x_hbm, src_hbm, dst_hbm, w_hbm, out_hbm,
          src_c, dst_c, w_c, rows, zbuf, acc, sem, semi):
    cid = lax.axis_index("c")
    sid = lax.axis_index("s")
    wid = cid * NS + sid

    zero16 = jnp.zeros((LANES,), jnp.float32)

    def zero_row(r, carry):
      for c in range(D // LANES):
        zbuf[r, pl.ds(c * LANES, LANES)] = zero16
      return carry

    lax.fori_loop(0, ZROWS, zero_row, 0)
    base = sid * ROWS_PER_TILE
    for k in range(ROWS_PER_TILE // ZROWS):
      pltpu.sync_copy(zbuf, acc.at[pl.ds(base + k * ZROWS, ZROWS)])
    plsc.subcore_barrier()

    def body(j, carry):
      cp1 = pltpu.async_copy(src_hbm.at[wid, j], src_c.at[0], semi)
      cp2 = pltpu.async_copy(dst_hbm.at[wid, j], dst_c.at[0], semi)
      cp3 = pltpu.async_copy(w_hbm.at[wid, pl.ds(j * SUB, SUB)], w_c, semi)
      cp1.wait()
      cp2.wait()
      cp3.wait()
      pltpu.async_copy(x_hbm.at[src_c.at[0]], rows, sem).wait()

      def scale16(i16, c2):
        w16 = w_c[pl.ds(i16 * LANES, LANES)]
        for bb in range(LANES):
          wspl = lax.gather(
              w16,
              jnp.full((LANES, 1), bb, jnp.int32),
              lax.GatherDimensionNumbers(
                  offset_dims=(), collapsed_slice_dims=(0,),
                  start_index_map=(0,)),
              slice_sizes=(1,),
              mode=lax.GatherScatterMode.PROMISE_IN_BOUNDS,
          )
          row = i16 * LANES + bb
          for c in range(D // LANES):
            rows[row, pl.ds(c * LANES, LANES)] = (
                rows[row, pl.ds(c * LANES, LANES)] * wspl
            )
        return c2

      lax.fori_loop(0, SUB // LANES, scale16, 0)
      pltpu.sync_copy(rows, acc.at[dst_c.at[0]], add=True)
      return carry

    lax.fori_loop(0, n_sub, body, 0)
    plsc.subcore_barrier()

    obufs = (zbuf, rows)
    for k in range(ROWS_PER_TILE // ZROWS):
      ob = obufs[k % 2]
      if k >= 2:
        pltpu.make_async_copy(
            ob, out_hbm.at[cid, pl.ds(base + (k - 2) * ZROWS, ZROWS)], semi
        ).wait()
      pltpu.sync_copy(acc.at[pl.ds(base + k * ZROWS, ZROWS)], ob)
      pltpu.async_copy(
          ob, out_hbm.at[cid, pl.ds(base + k * ZROWS, ZROWS)], semi
      )
    for k in range(ROWS_PER_TILE // ZROWS - 2, ROWS_PER_TILE // ZROWS):
      ob = obufs[k % 2]
      pltpu.make_async_copy(
          ob, out_hbm.at[cid, pl.ds(base + k * ZROWS, ZROWS)], semi
      ).wait()

  return agg(x, src, dst, w)


def _tc_finish(p, W):
  blk = 1000
  grid = (N // blk,)

  def body(p_ref, w_ref, o_ref):
    a = p_ref[0] + p_ref[1]
    h = jnp.dot(a, w_ref[...], preferred_element_type=jnp.float32)
    o_ref[...] = jnp.maximum(h, 0.0)

  return pl.pallas_call(
      body,
      grid=grid,
      in_specs=[
          pl.BlockSpec((NC, blk, D), lambda i: (0, i, 0)),
          pl.BlockSpec((D, D), lambda i: (0, 0)),
      ],
      out_specs=pl.BlockSpec((blk, D), lambda i: (i, 0)),
      out_shape=jax.ShapeDtypeStruct((N, D), jnp.float32),
  )(p, W)


@jax.jit
def kernel(x, edge_index, edge_weight, W):
  src = edge_index[0]
  dst = edge_index[1]
  e = src.shape[0]
  n_sub = -(-e // (NW * SUB))
  e_pad = NW * SUB * n_sub
  pad = e_pad - e
  src = jnp.concatenate([src, jnp.zeros((pad,), jnp.int32)]).reshape(NW, n_sub, SUB)
  dst = jnp.concatenate([dst, jnp.zeros((pad,), jnp.int32)]).reshape(NW, n_sub, SUB)
  w = jnp.concatenate([edge_weight, jnp.zeros((pad,), jnp.float32)]).reshape(
      NW, n_sub * SUB
  )
  p = _sc_aggregate(x, src, dst, w, n_sub)
  return _tc_finish(p, W)
